# Initial kernel scaffold; baseline (speedup 1.0000x reference)
import jax
import jax.numpy as jnp
from jax.experimental import pallas as pl


def kernel(seq, tok_table, pos_table):
    raise NotImplementedError("write your pallas kernel here")


# speedup vs baseline: 1.2747x; 1.2747x over previous
import jax
import jax.numpy as jnp
from jax import lax
from jax.experimental import pallas as pl
from jax.experimental.pallas import tpu as pltpu
from jax.experimental.pallas import tpu_sc as plsc

_NC = 2
_NS = 16
_NW = _NC * _NS
_LANES = 16
_IDX_W = 128


def _build(B, L, V, D):
    tokens = B * L
    chunk = tokens // _NW
    n_gather = chunk // _IDX_W

    mesh = plsc.VectorSubcoreMesh(
        core_axis_name="c", subcore_axis_name="s",
        num_cores=_NC, num_subcores=_NS,
    )

    def body(seq_hbm, tok_hbm, pos_hbm, out_hbm, idx_v, tok_v, pos_v, sem):
        c = lax.axis_index("c")
        s = lax.axis_index("s")
        wid = s * _NC + c
        base = wid * chunk

        pltpu.sync_copy(seq_hbm.at[pl.ds(wid * n_gather, n_gather)], idx_v)

        for j in range(n_gather):
            pltpu.async_copy(
                tok_hbm.at[idx_v.at[j]],
                tok_v.at[pl.ds(j * _IDX_W, _IDX_W)],
                sem,
            )

        pos_base = lax.rem(base, L)
        pltpu.sync_copy(pos_hbm.at[pl.ds(pos_base, chunk)], pos_v)

        for j in range(n_gather):
            pltpu.make_async_copy(
                tok_hbm.at[idx_v.at[j]],
                tok_v.at[pl.ds(j * _IDX_W, _IDX_W)],
                sem,
            ).wait()

        def row(i, carry):
            for cb in range(D // _LANES):
                sl = pl.ds(cb * _LANES, _LANES)
                tok_v[i, sl] = tok_v[i, sl] + pos_v[i, sl]
            return carry

        lax.fori_loop(0, chunk, row, 0)

        pltpu.sync_copy(tok_v, out_hbm.at[pl.ds(base, chunk)])

    kern = pl.kernel(
        body,
        out_type=jax.ShapeDtypeStruct((tokens, D), jnp.float32),
        mesh=mesh,
        scratch_types=[
            pltpu.VMEM((n_gather, _IDX_W), jnp.int32),
            pltpu.VMEM((chunk, D), jnp.float32),
            pltpu.VMEM((chunk, D), jnp.float32),
            pltpu.SemaphoreType.DMA,
        ],
    )
    return kern


def kernel(seq, tok_table, pos_table):
    B, L = seq.shape
    V, D = tok_table.shape
    seq_flat = seq.astype(jnp.int32).reshape(B * L // _IDX_W, _IDX_W)
    out = _build(B, L, V, D)(seq_flat, tok_table, pos_table)
    return out.reshape(B, L, D)
